# baseline (speedup 1.0000x reference)
import functools

import jax
import jax.numpy as jnp
from jax.experimental import pallas as pl
from jax.experimental.pallas import tpu as pltpu

D_IN = 2048
D_FEATURES = 4096
BLOCK_SIZE = 1024
NUM_BLOCKS = 4
TILE_R = 512


def _body(x_ref, ih_ref, sh_ref, o_ref, p_ref):
    @pl.when(pl.program_id(0) == 0)
    def _build_projection():
        cols = jax.lax.broadcasted_iota(jnp.int32, (D_IN, BLOCK_SIZE), 1)
        for i in range(NUM_BLOCKS):
            ih = ih_ref[:, i : i + 1]
            sh = sh_ref[:, i : i + 1]
            p_ref[:, i * BLOCK_SIZE : (i + 1) * BLOCK_SIZE] = jnp.where(
                cols == ih, sh, 0.0
            ).astype(jnp.bfloat16)

    acc = jax.lax.dot_general(
        x_ref[...],
        p_ref[...],
        (((1,), (0,)), ((), ())),
        preferred_element_type=jnp.float32,
    )
    o_ref[...] = acc * 0.5


@jax.jit
def kernel(x, i_hash, s_hash):
    rows = x.shape[0]
    xb = x.astype(jnp.bfloat16)
    ih_t = i_hash.T
    sh_t = s_hash.T
    grid = (rows // TILE_R,)
    out = pl.pallas_call(
        _body,
        grid=grid,
        in_specs=[
            pl.BlockSpec((TILE_R, D_IN), lambda r: (r, 0)),
            pl.BlockSpec((D_IN, NUM_BLOCKS), lambda r: (0, 0)),
            pl.BlockSpec((D_IN, NUM_BLOCKS), lambda r: (0, 0)),
        ],
        out_specs=pl.BlockSpec((TILE_R, D_FEATURES), lambda r: (r, 0)),
        out_shape=jax.ShapeDtypeStruct((rows, D_FEATURES), jnp.float32),
        scratch_shapes=[pltpu.VMEM((D_IN, D_FEATURES), jnp.bfloat16)],
    )(xb, ih_t, sh_t)
    return out


# speedup vs baseline: 1.1843x; 1.1843x over previous
import functools

import jax
import jax.numpy as jnp
from jax.experimental import pallas as pl
from jax.experimental.pallas import tpu as pltpu

D_IN = 2048
D_FEATURES = 4096
BLOCK_SIZE = 1024
NUM_BLOCKS = 4
TILE_R = 256


def _body(x_ref, ih_ref, sh_ref, o_ref, p_ref):
    @pl.when(pl.program_id(0) == 0)
    def _build_projection():
        cols = jax.lax.broadcasted_iota(jnp.int32, (D_IN, BLOCK_SIZE), 1)
        for i in range(NUM_BLOCKS):
            ih = ih_ref[:, i : i + 1]
            sh = sh_ref[:, i : i + 1]
            p_ref[:, i * BLOCK_SIZE : (i + 1) * BLOCK_SIZE] = jnp.where(
                cols == ih, sh, 0.0
            ).astype(jnp.bfloat16)

    xb = x_ref[...].astype(jnp.bfloat16)
    acc = jax.lax.dot_general(
        xb,
        p_ref[...],
        (((1,), (0,)), ((), ())),
        preferred_element_type=jnp.float32,
    )
    o_ref[...] = acc * 0.5


@jax.jit
def kernel(x, i_hash, s_hash):
    rows = x.shape[0]
    ih_t = i_hash.T
    sh_t = s_hash.T
    grid = (rows // TILE_R,)
    out = pl.pallas_call(
        _body,
        grid=grid,
        in_specs=[
            pl.BlockSpec((TILE_R, D_IN), lambda r: (r, 0)),
            pl.BlockSpec((D_IN, NUM_BLOCKS), lambda r: (0, 0)),
            pl.BlockSpec((D_IN, NUM_BLOCKS), lambda r: (0, 0)),
        ],
        out_specs=pl.BlockSpec((TILE_R, D_FEATURES), lambda r: (r, 0)),
        out_shape=jax.ShapeDtypeStruct((rows, D_FEATURES), jnp.float32),
        scratch_shapes=[pltpu.VMEM((D_IN, D_FEATURES), jnp.bfloat16)],
    )(x, ih_t, sh_t)
    return out


# speedup vs baseline: 1.1990x; 1.0124x over previous
import functools

import jax
import jax.numpy as jnp
from jax.experimental import pallas as pl
from jax.experimental.pallas import tpu as pltpu

D_IN = 2048
D_FEATURES = 4096
BLOCK_SIZE = 1024
NUM_BLOCKS = 4
TILE_R = 512


def _body(x_ref, ih_ref, sh_ref, o_ref, p_ref):
    @pl.when(pl.program_id(0) == 0)
    def _build_projection():
        cols = jax.lax.broadcasted_iota(jnp.int32, (D_IN, BLOCK_SIZE), 1)
        for i in range(NUM_BLOCKS):
            ih = ih_ref[:, i : i + 1]
            sh = sh_ref[:, i : i + 1] * 0.5
            p_ref[:, i * BLOCK_SIZE : (i + 1) * BLOCK_SIZE] = jnp.where(
                cols == ih, sh, 0.0
            ).astype(jnp.bfloat16)

    xb = x_ref[...].astype(jnp.bfloat16)
    acc = jax.lax.dot_general(
        xb,
        p_ref[...],
        (((1,), (0,)), ((), ())),
        preferred_element_type=jnp.float32,
    )
    o_ref[...] = acc


@jax.jit
def kernel(x, i_hash, s_hash):
    rows = x.shape[0]
    ih_t = i_hash.T
    sh_t = s_hash.T
    grid = (rows // TILE_R,)
    out = pl.pallas_call(
        _body,
        grid=grid,
        in_specs=[
            pl.BlockSpec((TILE_R, D_IN), lambda r: (r, 0)),
            pl.BlockSpec((D_IN, NUM_BLOCKS), lambda r: (0, 0)),
            pl.BlockSpec((D_IN, NUM_BLOCKS), lambda r: (0, 0)),
        ],
        out_specs=pl.BlockSpec((TILE_R, D_FEATURES), lambda r: (r, 0)),
        out_shape=jax.ShapeDtypeStruct((rows, D_FEATURES), jnp.float32),
        scratch_shapes=[pltpu.VMEM((D_IN, D_FEATURES), jnp.bfloat16)],
    )(x, ih_t, sh_t)
    return out
